# baseline (speedup 1.0000x reference)
import jax
import jax.numpy as jnp
from jax.experimental import pallas as pl
from jax.experimental.pallas import tpu as pltpu


def _copy_body(src_ref, dst_ref, sem):
    cp = pltpu.make_async_copy(src_ref, dst_ref, sem)
    cp.start()
    cp.wait()


def kernel(inputs, weight):
    bsz, seq_len = inputs.shape[:2]
    return pl.pallas_call(
        _copy_body,
        out_shape=jax.ShapeDtypeStruct((seq_len, weight.shape[1]), weight.dtype),
        in_specs=[pl.BlockSpec(memory_space=pl.ANY)],
        out_specs=pl.BlockSpec(memory_space=pl.ANY),
        scratch_shapes=[pltpu.SemaphoreType.DMA],
    )(weight)
